# baseline (speedup 1.0000x reference)
import jax
import jax.numpy as jnp
from jax.experimental import pallas as pl
from jax.experimental.pallas import tpu as pltpu

_B, _M, _H, _W = 32, 4, 256, 256
_HB = 256


def _hom_kernel(idx_ref, g_ref, t_ref, o_ref):
    b = pl.program_id(0)
    m = pl.program_id(1)
    i = idx_ref[b]
    x = g_ref[0, 0, 0]
    y = g_ref[0, 0, 1]
    t = [t_ref[i, m, k] for k in range(8)]
    den = t[6] * x + t[7] * y + 1.0
    r = jax.lax.reciprocal(den)
    r = r * (2.0 - den * r)
    nx = (t[0] * x + t[1] * y + t[2]) * r
    ny = (t[3] * x + t[4] * y + t[5]) * r
    o_ref[0, 0, 0::4, :] = nx[:, :128]
    o_ref[0, 0, 1::4, :] = ny[:, :128]
    o_ref[0, 0, 2::4, :] = nx[:, 128:]
    o_ref[0, 0, 3::4, :] = ny[:, 128:]


def kernel(idx, grid, theta):
    n_frames = theta.shape[0]
    gp = jnp.transpose(grid, (0, 1, 4, 2, 3))
    out = pl.pallas_call(
        _hom_kernel,
        grid_spec=pltpu.PrefetchScalarGridSpec(
            num_scalar_prefetch=1,
            grid=(_B, _M, _H // _HB),
            in_specs=[
                pl.BlockSpec((1, 1, 2, _HB, _W), lambda b, m, h, i_ref: (b, m, 0, h, 0)),
                pl.BlockSpec((n_frames, _M, 8), lambda b, m, h, i_ref: (0, 0, 0)),
            ],
            out_specs=pl.BlockSpec((1, 1, 4 * _HB, 128), lambda b, m, h, i_ref: (b, m, h, 0)),
        ),
        out_shape=jax.ShapeDtypeStruct((_B, _M, 4 * _H, 128), jnp.float32),
        compiler_params=pltpu.CompilerParams(
            dimension_semantics=("parallel", "parallel", "arbitrary"),
        ),
    )(idx, gp, theta)
    o6 = out.reshape(_B, _M, _H, 2, 2, 128)
    return jnp.transpose(o6, (0, 1, 2, 3, 5, 4)).reshape(_B, _M, _H, _W, 2)


# speedup vs baseline: 1.8101x; 1.8101x over previous
import jax
import jax.numpy as jnp
from jax.experimental import pallas as pl
from jax.experimental.pallas import tpu as pltpu

_B, _M, _H, _W = 32, 4, 256, 256
_HB = 256


def _hom_kernel(idx_ref, g_ref, t_ref, o_ref):
    b = pl.program_id(0)
    i = idx_ref[b]
    for m in range(_M):
        x = g_ref[0, m, 0]
        y = g_ref[0, m, 1]
        t = [t_ref[i, m, k] for k in range(8)]
        den = t[6] * x + t[7] * y + 1.0
        r = jax.lax.reciprocal(den)
        r = r * (2.0 - den * r)
        nx = (t[0] * x + t[1] * y + t[2]) * r
        ny = (t[3] * x + t[4] * y + t[5]) * r
        o_ref[0, m, 0::4, :] = nx[:, :128]
        o_ref[0, m, 1::4, :] = ny[:, :128]
        o_ref[0, m, 2::4, :] = nx[:, 128:]
        o_ref[0, m, 3::4, :] = ny[:, 128:]


def kernel(idx, grid, theta):
    n_frames = theta.shape[0]
    gp = jnp.transpose(grid, (0, 1, 4, 2, 3))
    out = pl.pallas_call(
        _hom_kernel,
        grid_spec=pltpu.PrefetchScalarGridSpec(
            num_scalar_prefetch=1,
            grid=(_B, _H // _HB),
            in_specs=[
                pl.BlockSpec((1, _M, 2, _HB, _W), lambda b, h, i_ref: (b, 0, 0, h, 0)),
                pl.BlockSpec((n_frames, _M, 8), lambda b, h, i_ref: (0, 0, 0)),
            ],
            out_specs=pl.BlockSpec((1, _M, 4 * _HB, 128), lambda b, h, i_ref: (b, 0, h, 0)),
        ),
        out_shape=jax.ShapeDtypeStruct((_B, _M, 4 * _H, 128), jnp.float32),
        compiler_params=pltpu.CompilerParams(
            dimension_semantics=("parallel", "arbitrary"),
        ),
    )(idx, gp, theta)
    o6 = out.reshape(_B, _M, _H, 2, 2, 128)
    return jnp.transpose(o6, (0, 1, 2, 3, 5, 4)).reshape(_B, _M, _H, _W, 2)


# speedup vs baseline: 2.1035x; 1.1621x over previous
import jax
import jax.numpy as jnp
from jax.experimental import pallas as pl
from jax.experimental.pallas import tpu as pltpu

_B, _M, _H, _W = 32, 4, 256, 256
_HB = 256


_BB = 2


def _hom_kernel(idx_ref, g_ref, t_ref, o_ref):
    b0 = pl.program_id(0) * _BB
    for bb in range(_BB):
        i = idx_ref[b0 + bb]
        for m in range(_M):
            x = g_ref[bb, m, 0]
            y = g_ref[bb, m, 1]
            t = [t_ref[i, m, k] for k in range(8)]
            den = t[6] * x + t[7] * y + 1.0
            r = jax.lax.reciprocal(den)
            r = r * (2.0 - den * r)
            nx = (t[0] * x + t[1] * y + t[2]) * r
            ny = (t[3] * x + t[4] * y + t[5]) * r
            o_ref[bb, m, 0::4, :] = nx[:, :128]
            o_ref[bb, m, 1::4, :] = ny[:, :128]
            o_ref[bb, m, 2::4, :] = nx[:, 128:]
            o_ref[bb, m, 3::4, :] = ny[:, 128:]


def kernel(idx, grid, theta):
    n_frames = theta.shape[0]
    gp = jnp.transpose(grid, (0, 1, 4, 2, 3))
    out = pl.pallas_call(
        _hom_kernel,
        grid_spec=pltpu.PrefetchScalarGridSpec(
            num_scalar_prefetch=1,
            grid=(_B // _BB, _H // _HB),
            in_specs=[
                pl.BlockSpec((_BB, _M, 2, _HB, _W), lambda b, h, i_ref: (b, 0, 0, h, 0)),
                pl.BlockSpec((n_frames, _M, 8), lambda b, h, i_ref: (0, 0, 0)),
            ],
            out_specs=pl.BlockSpec((_BB, _M, 4 * _HB, 128), lambda b, h, i_ref: (b, 0, h, 0)),
        ),
        out_shape=jax.ShapeDtypeStruct((_B, _M, 4 * _H, 128), jnp.float32),
        compiler_params=pltpu.CompilerParams(
            dimension_semantics=("parallel", "arbitrary"),
        ),
    )(idx, gp, theta)
    o6 = out.reshape(_B, _M, _H, 2, 2, 128)
    return jnp.transpose(o6, (0, 1, 2, 3, 5, 4)).reshape(_B, _M, _H, _W, 2)


# speedup vs baseline: 2.1937x; 1.0429x over previous
import jax
import jax.numpy as jnp
from jax.experimental import pallas as pl
from jax.experimental.pallas import tpu as pltpu

_B, _M, _H, _W = 32, 4, 256, 256
_HB = 256


_BB = 4


def _hom_kernel(idx_ref, g_ref, t_ref, o_ref):
    b0 = pl.program_id(0) * _BB
    for bb in range(_BB):
        i = idx_ref[b0 + bb]
        for m in range(_M):
            x = g_ref[bb, m, 0]
            y = g_ref[bb, m, 1]
            t = [t_ref[i, m, k] for k in range(8)]
            den = t[6] * x + t[7] * y + 1.0
            r = jax.lax.reciprocal(den)
            r = r * (2.0 - den * r)
            nx = (t[0] * x + t[1] * y + t[2]) * r
            ny = (t[3] * x + t[4] * y + t[5]) * r
            o_ref[bb, m, 0::4, :] = nx[:, :128]
            o_ref[bb, m, 1::4, :] = ny[:, :128]
            o_ref[bb, m, 2::4, :] = nx[:, 128:]
            o_ref[bb, m, 3::4, :] = ny[:, 128:]


def kernel(idx, grid, theta):
    n_frames = theta.shape[0]
    gp = jnp.transpose(grid, (0, 1, 4, 2, 3))
    out = pl.pallas_call(
        _hom_kernel,
        grid_spec=pltpu.PrefetchScalarGridSpec(
            num_scalar_prefetch=1,
            grid=(_B // _BB, _H // _HB),
            in_specs=[
                pl.BlockSpec((_BB, _M, 2, _HB, _W), lambda b, h, i_ref: (b, 0, 0, h, 0)),
                pl.BlockSpec((n_frames, _M, 8), lambda b, h, i_ref: (0, 0, 0)),
            ],
            out_specs=pl.BlockSpec((_BB, _M, 4 * _HB, 128), lambda b, h, i_ref: (b, 0, h, 0)),
        ),
        out_shape=jax.ShapeDtypeStruct((_B, _M, 4 * _H, 128), jnp.float32),
        compiler_params=pltpu.CompilerParams(
            dimension_semantics=("parallel", "arbitrary"),
        ),
    )(idx, gp, theta)
    o6 = out.reshape(_B, _M, _H, 2, 2, 128)
    return jnp.transpose(o6, (0, 1, 2, 3, 5, 4)).reshape(_B, _M, _H, _W, 2)


# speedup vs baseline: 2.2121x; 1.0084x over previous
import jax
import jax.numpy as jnp
from jax.experimental import pallas as pl
from jax.experimental.pallas import tpu as pltpu

_B, _M, _H, _W = 32, 4, 256, 256
_HB = 256


_BB = 4


def _hom_kernel(idx_ref, g_ref, t_ref, o_ref):
    b0 = pl.program_id(0) * _BB
    for bb in range(_BB):
        i = idx_ref[b0 + bb]
        for m in range(_M):
            x = g_ref[bb, m, 0]
            y = g_ref[bb, m, 1]
            t = [t_ref[i, m, k] for k in range(8)]
            den = t[6] * x + t[7] * y + 1.0
            r = jax.lax.reciprocal(den)
            nx = (t[0] * x + t[1] * y + t[2]) * r
            ny = (t[3] * x + t[4] * y + t[5]) * r
            o_ref[bb, m, 0::4, :] = nx[:, :128]
            o_ref[bb, m, 1::4, :] = ny[:, :128]
            o_ref[bb, m, 2::4, :] = nx[:, 128:]
            o_ref[bb, m, 3::4, :] = ny[:, 128:]


def kernel(idx, grid, theta):
    n_frames = theta.shape[0]
    gp = jnp.transpose(grid, (0, 1, 4, 2, 3))
    out = pl.pallas_call(
        _hom_kernel,
        grid_spec=pltpu.PrefetchScalarGridSpec(
            num_scalar_prefetch=1,
            grid=(_B // _BB, _H // _HB),
            in_specs=[
                pl.BlockSpec((_BB, _M, 2, _HB, _W), lambda b, h, i_ref: (b, 0, 0, h, 0)),
                pl.BlockSpec((n_frames, _M, 8), lambda b, h, i_ref: (0, 0, 0)),
            ],
            out_specs=pl.BlockSpec((_BB, _M, 4 * _HB, 128), lambda b, h, i_ref: (b, 0, h, 0)),
        ),
        out_shape=jax.ShapeDtypeStruct((_B, _M, 4 * _H, 128), jnp.float32),
        compiler_params=pltpu.CompilerParams(
            dimension_semantics=("parallel", "arbitrary"),
            vmem_limit_bytes=100 * 1024 * 1024,
        ),
    )(idx, gp, theta)
    o6 = out.reshape(_B, _M, _H, 2, 2, 128)
    return jnp.transpose(o6, (0, 1, 2, 3, 5, 4)).reshape(_B, _M, _H, _W, 2)
